# Initial kernel scaffold; baseline (speedup 1.0000x reference)
import jax
import jax.numpy as jnp
from jax.experimental import pallas as pl


def kernel(x, edge_index, W_l1, W_r1, att1, b1, W_l2, W_r2, att2, b2):
    raise NotImplementedError("write your pallas kernel here")


# speedup vs baseline: 21.6684x; 21.6684x over previous
import functools

import jax
import jax.numpy as jnp
from jax import lax
from jax.experimental import pallas as pl
from jax.experimental.pallas import tpu as pltpu
from jax.experimental.pallas import tpu_sc as plsc

N, E, D = 10000, 320000, 128
NPAD = 10240
ET = E + N
NC, NS, L = 2, 16, 16
NW = NC * NS
K = 64
EW = 10368
CH = EW // K
ET_PAD = NW * EW
ROWS_PER_TILE = NPAD // NS
NPD8 = NPAD // 8
DRPT = NPD8 // NS

BR = 1024


def _mm2_body(x_ref, wl_ref, wr_ref, xl_ref, xr_ref):
    xb = x_ref[...]
    xl_ref[...] = jnp.dot(xb, wl_ref[...], preferred_element_type=jnp.float32)
    xr_ref[...] = jnp.dot(xb, wr_ref[...], preferred_element_type=jnp.float32)


def _mm2(xp, wl, wr):
    return pl.pallas_call(
        _mm2_body,
        grid=(NPAD // BR,),
        in_specs=[pl.BlockSpec((BR, D), lambda i: (i, 0)),
                  pl.BlockSpec((D, D), lambda i: (0, 0)),
                  pl.BlockSpec((D, D), lambda i: (0, 0))],
        out_specs=[pl.BlockSpec((BR, D), lambda i: (i, 0)),
                   pl.BlockSpec((BR, D), lambda i: (i, 0))],
        out_shape=[jax.ShapeDtypeStruct((NPAD, D), jnp.float32),
                   jax.ShapeDtypeStruct((NPAD, D), jnp.float32)],
    )(xp, wl, wr)


def _combine1_body(val_ref, den_ref, b_ref, wl_ref, wr_ref, xl_ref, xr_ref):
    v = val_ref[0] + val_ref[1]
    d = den_ref[0] + den_ref[1]
    cw = D // 4
    parts = [v[:, h * cw:(h + 1) * cw] / (d[:, h:h + 1] + 1e-16)
             for h in range(4)]
    h1 = jnp.concatenate(parts, axis=1) + b_ref[...]
    act = jnp.maximum(h1, 0.0) + jnp.log1p(jnp.exp(-jnp.abs(h1)))
    xl_ref[...] = jnp.dot(act, wl_ref[...], preferred_element_type=jnp.float32)
    xr_ref[...] = jnp.dot(act, wr_ref[...], preferred_element_type=jnp.float32)


def _combine1(val, den, b, wl, wr):
    return pl.pallas_call(
        _combine1_body,
        grid=(NPAD // BR,),
        in_specs=[pl.BlockSpec((2, BR, D), lambda i: (0, i, 0)),
                  pl.BlockSpec((2, BR, L), lambda i: (0, i, 0)),
                  pl.BlockSpec((1, D), lambda i: (0, 0)),
                  pl.BlockSpec((D, D), lambda i: (0, 0)),
                  pl.BlockSpec((D, D), lambda i: (0, 0))],
        out_specs=[pl.BlockSpec((BR, D), lambda i: (i, 0)),
                   pl.BlockSpec((BR, D), lambda i: (i, 0))],
        out_shape=[jax.ShapeDtypeStruct((NPAD, D), jnp.float32),
                   jax.ShapeDtypeStruct((NPAD, D), jnp.float32)],
    )(val, den, b.reshape(1, D), wl, wr)


def _final_body(val_ref, den_ref, b_ref, out_ref):
    v = val_ref[0] + val_ref[1]
    d = den_ref[0] + den_ref[1]
    out_ref[...] = v / (d[:, 0:1] + 1e-16) + b_ref[...]


def _final(val, den, b):
    return pl.pallas_call(
        _final_body,
        grid=(NPAD // BR,),
        in_specs=[pl.BlockSpec((2, BR, D), lambda i: (0, i, 0)),
                  pl.BlockSpec((2, BR, L), lambda i: (0, i, 0)),
                  pl.BlockSpec((1, D), lambda i: (0, 0))],
        out_specs=pl.BlockSpec((BR, D), lambda i: (i, 0)),
        out_shape=jax.ShapeDtypeStruct((NPAD, D), jnp.float32),
    )(val, den, b.reshape(1, D))


def _make_sc_layer(H):
    C = D // H
    mesh = plsc.VectorSubcoreMesh(core_axis_name="c", subcore_axis_name="s",
                                  num_cores=NC, num_subcores=NS)

    def body(xl_hbm, xr_hbm, src_hbm, dst_hbm, att_hbm, val_hbm, den_hbm,
             srcb, dstb, dstb2, xlr, xrr, sp, sp2, attv, accval, accden):
        ci = lax.axis_index("c")
        si = lax.axis_index("s")
        wid = si * NC + ci

        zero16 = jnp.zeros((L,), jnp.float32)

        def zrow(r, _):
            for v in range(D // L):
                xlr[r, pl.ds(v * L, L)] = zero16
                sp2[r, pl.ds(v * L, L)] = zero16
            sp[r] = zero16
            return 0
        lax.fori_loop(0, K, zrow, 0)
        for i in range(ROWS_PER_TILE // K):
            r0 = si * ROWS_PER_TILE + i * K
            pltpu.sync_copy(xlr, accval.at[pl.ds(r0, K)])
        pltpu.sync_copy(sp2, accden.at[pl.ds(si * DRPT, K)])
        pltpu.sync_copy(sp2.at[pl.ds(0, DRPT - K)],
                        accden.at[pl.ds(si * DRPT + K, DRPT - K)])
        pltpu.sync_copy(att_hbm, attv)
        plsc.subcore_barrier()

        att_vecs = [attv[pl.ds(k * L, L)] for k in range(D // L)]
        lane = lax.iota(jnp.int32, L)
        NV = D // L
        VPH = NV // H

        def chunk(j, _):
            base = wid * EW + j * K
            pltpu.sync_copy(src_hbm.at[pl.ds(base, K)], srcb)
            pltpu.sync_copy(dst_hbm.at[pl.ds(base, K)], dstb)
            pltpu.sync_copy(xl_hbm.at[srcb], xlr)
            pltpu.sync_copy(xr_hbm.at[dstb], xrr)

            def erow(e, _):
                xs = [xlr[e, pl.ds(v * L, L)] for v in range(NV)]
                rs = [xrr[e, pl.ds(v * L, L)] for v in range(NV)]
                lvec = zero16
                for h in range(H):
                    hs = zero16
                    for v in range(h * VPH, (h + 1) * VPH):
                        s = xs[v] + rs[v]
                        hs = hs + jnp.maximum(s, 0.2 * s) * att_vecs[v]
                    logit = jnp.sum(hs)
                    lvec = lvec + jnp.where(lane == h, logit, 0.0)
                pvec = jnp.where(lane < H, jnp.exp(lvec), 0.0)
                sp[e] = pvec
                for v in range(NV):
                    xlr[e, pl.ds(v * L, L)] = xs[v] * pvec[v // VPH]
                return 0
            lax.fori_loop(0, K, erow, 0)

            def grp(g, _):
                eids = lane + g * L
                dvec = dstb[pl.ds(g * L, L)]
                dstb2[pl.ds(g * L, L)] = lax.shift_right_logical(dvec, 3)
                cols0 = jnp.bitwise_and(dvec, 7) * L
                for h in range(H):
                    ph = plsc.load_gather(sp, [eids, jnp.full((L,), h, jnp.int32)])
                    plsc.store_scatter(sp2, [eids, cols0 + h], ph)
                return 0
            lax.fori_loop(0, K // L, grp, 0)

            pltpu.sync_copy(xlr, accval.at[dstb], add=True)
            pltpu.sync_copy(sp2, accden.at[dstb2], add=True)

            def gz(g, _):
                eids = lane + g * L
                dvec = dstb[pl.ds(g * L, L)]
                cols0 = jnp.bitwise_and(dvec, 7) * L
                for h in range(H):
                    plsc.store_scatter(sp2, [eids, cols0 + h], zero16)
                return 0
            lax.fori_loop(0, K // L, gz, 0)
            return 0
        lax.fori_loop(0, CH, chunk, 0)

        plsc.subcore_barrier()
        r0 = si * ROWS_PER_TILE
        pltpu.sync_copy(accval.at[pl.ds(r0, ROWS_PER_TILE)],
                        val_hbm.at[ci, pl.ds(r0, ROWS_PER_TILE)])
        d0 = si * DRPT
        pltpu.sync_copy(accden.at[pl.ds(d0, DRPT)],
                        den_hbm.at[ci, pl.ds(d0, DRPT)])

    return pl.kernel(
        body,
        out_type=(jax.ShapeDtypeStruct((NC, NPAD, D), jnp.float32),
                  jax.ShapeDtypeStruct((NC, NPD8, D), jnp.float32)),
        mesh=mesh,
        compiler_params=pltpu.CompilerParams(needs_layout_passes=False),
        scratch_types=(
            pltpu.VMEM((K,), jnp.int32),
            pltpu.VMEM((K,), jnp.int32),
            pltpu.VMEM((K,), jnp.int32),
            pltpu.VMEM((K, D), jnp.float32),
            pltpu.VMEM((K, D), jnp.float32),
            pltpu.VMEM((K, L), jnp.float32),
            pltpu.VMEM((K, D), jnp.float32),
            pltpu.VMEM((D,), jnp.float32),
            pltpu.VMEM_SHARED((NPAD, D), jnp.float32),
            pltpu.VMEM_SHARED((NPD8, D), jnp.float32),
        ),
    )


@functools.lru_cache(maxsize=None)
def _sc_layer(H):
    return _make_sc_layer(H)


def kernel(x, edge_index, W_l1, W_r1, att1, b1, W_l2, W_r2, att2, b2):
    xpad = jnp.zeros((NPAD, D), jnp.float32).at[:N].set(x)
    loop = jnp.arange(N, dtype=jnp.int32)
    pad = jnp.full((ET_PAD - ET,), NPAD - 1, jnp.int32)
    src = jnp.concatenate([edge_index[0].astype(jnp.int32), loop, pad])
    dst = jnp.concatenate([edge_index[1].astype(jnp.int32), loop, pad])

    xl1, xr1 = _mm2(xpad, W_l1, W_r1)
    val1, den1 = _sc_layer(4)(xl1, xr1, src, dst, att1.reshape(D))
    xl2, xr2 = _combine1(val1, den1.reshape(NC, NPAD, L), b1, W_l2, W_r2)
    val2, den2 = _sc_layer(1)(xl2, xr2, src, dst, att2.reshape(D))
    out = _final(val2, den2.reshape(NC, NPAD, L), b2)
    return out[:N]


# speedup vs baseline: 56.4841x; 2.6068x over previous
import functools

import jax
import jax.numpy as jnp
from jax import lax
from jax.experimental import pallas as pl
from jax.experimental.pallas import tpu as pltpu
from jax.experimental.pallas import tpu_sc as plsc

N, E, D = 10000, 320000, 128
NPAD = 10240
ET = E + N
NC, NS, L = 2, 16, 16
NW = NC * NS
K = 32
EW = 10368
CH = EW // K
ET_PAD = NW * EW
ROWS_PER_TILE = NPAD // NS
NPD8 = NPAD // 8
DRPT = NPD8 // NS

BR = 1024


def _mm2_body(x_ref, wl_ref, wr_ref, xl_ref, xr_ref):
    xb = x_ref[...]
    xl_ref[...] = jnp.dot(xb, wl_ref[...], preferred_element_type=jnp.float32)
    xr_ref[...] = jnp.dot(xb, wr_ref[...], preferred_element_type=jnp.float32)


def _mm2(xp, wl, wr):
    return pl.pallas_call(
        _mm2_body,
        grid=(NPAD // BR,),
        in_specs=[pl.BlockSpec((BR, D), lambda i: (i, 0)),
                  pl.BlockSpec((D, D), lambda i: (0, 0)),
                  pl.BlockSpec((D, D), lambda i: (0, 0))],
        out_specs=[pl.BlockSpec((BR, D), lambda i: (i, 0)),
                   pl.BlockSpec((BR, D), lambda i: (i, 0))],
        out_shape=[jax.ShapeDtypeStruct((NPAD, D), jnp.float32),
                   jax.ShapeDtypeStruct((NPAD, D), jnp.float32)],
    )(xp, wl, wr)


def _combine1_body(val_ref, den_ref, b_ref, wl_ref, wr_ref, xl_ref, xr_ref):
    v = val_ref[0] + val_ref[1]
    d = den_ref[0] + den_ref[1]
    cw = D // 4
    parts = [v[:, h * cw:(h + 1) * cw] / (d[:, h:h + 1] + 1e-16)
             for h in range(4)]
    h1 = jnp.concatenate(parts, axis=1) + b_ref[...]
    act = jnp.maximum(h1, 0.0) + jnp.log1p(jnp.exp(-jnp.abs(h1)))
    xl_ref[...] = jnp.dot(act, wl_ref[...], preferred_element_type=jnp.float32)
    xr_ref[...] = jnp.dot(act, wr_ref[...], preferred_element_type=jnp.float32)


def _combine1(val, den, b, wl, wr):
    return pl.pallas_call(
        _combine1_body,
        grid=(NPAD // BR,),
        in_specs=[pl.BlockSpec((2, BR, D), lambda i: (0, i, 0)),
                  pl.BlockSpec((2, BR, L), lambda i: (0, i, 0)),
                  pl.BlockSpec((1, D), lambda i: (0, 0)),
                  pl.BlockSpec((D, D), lambda i: (0, 0)),
                  pl.BlockSpec((D, D), lambda i: (0, 0))],
        out_specs=[pl.BlockSpec((BR, D), lambda i: (i, 0)),
                   pl.BlockSpec((BR, D), lambda i: (i, 0))],
        out_shape=[jax.ShapeDtypeStruct((NPAD, D), jnp.float32),
                   jax.ShapeDtypeStruct((NPAD, D), jnp.float32)],
    )(val, den, b.reshape(1, D), wl, wr)


def _final_body(val_ref, den_ref, b_ref, out_ref):
    v = val_ref[0] + val_ref[1]
    d = den_ref[0] + den_ref[1]
    out_ref[...] = v / (d[:, 0:1] + 1e-16) + b_ref[...]


def _final(val, den, b):
    return pl.pallas_call(
        _final_body,
        grid=(NPAD // BR,),
        in_specs=[pl.BlockSpec((2, BR, D), lambda i: (0, i, 0)),
                  pl.BlockSpec((2, BR, L), lambda i: (0, i, 0)),
                  pl.BlockSpec((1, D), lambda i: (0, 0))],
        out_specs=pl.BlockSpec((BR, D), lambda i: (i, 0)),
        out_shape=jax.ShapeDtypeStruct((NPAD, D), jnp.float32),
    )(val, den, b.reshape(1, D))


def _make_sc_layer(H):
    C = D // H
    mesh = plsc.VectorSubcoreMesh(core_axis_name="c", subcore_axis_name="s",
                                  num_cores=NC, num_subcores=NS)

    def body(xl_hbm, xr_hbm, src_hbm, dst_hbm, att_hbm, val_hbm, den_hbm,
             srcb0, srcb1, dstb0, dstb1, xlr0, xlr1, xrr0, xrr1,
             sidx0, sidx1, czb0, czb1, wbuf, sp, attv, acc,
             semidx0, semidx1, semrow0, semrow1, semscat):
        ci = lax.axis_index("c")
        si = lax.axis_index("s")
        wid = si * NC + ci

        zero16 = jnp.zeros((L,), jnp.float32)
        srcb = (srcb0, srcb1)
        dstb = (dstb0, dstb1)
        xlr = (xlr0, xlr1)
        xrr = (xrr0, xrr1)
        sidx = (sidx0, sidx1)
        czb = (czb0, czb1)
        semidx = (semidx0, semidx1)
        semrow = (semrow0, semrow1)

        def zrow(r, _):
            for v in range(D // L):
                wbuf[r, pl.ds(v * L, L)] = zero16
            return 0
        lax.fori_loop(0, 2 * K, zrow, 0)
        def zsp(r, _):
            sp[r] = zero16
            return 0
        lax.fori_loop(0, K, zsp, 0)
        for i in range(12):
            r0 = si * 720 + i * 60
            pltpu.sync_copy(wbuf.at[pl.ds(0, 60)], acc.at[pl.ds(r0, 60)])
        pltpu.sync_copy(att_hbm, attv)

        att_vecs = [attv[pl.ds(k * L, L)] for k in range(D // L)]
        lane = lax.iota(jnp.int32, L)
        NV = D // L
        VPH = NV // H

        def idx_issue(jn, s):
            base = wid * EW + jn * K
            pltpu.async_copy(src_hbm.at[pl.ds(base, K)], srcb[s], semidx[s])
            pltpu.async_copy(dst_hbm.at[pl.ds(base, K)], dstb[s], semidx[s])

        def rows_issue(s):
            pltpu.make_async_copy(src_hbm.at[pl.ds(0, K)], srcb[s],
                                  semidx[s]).wait()
            pltpu.make_async_copy(dst_hbm.at[pl.ds(0, K)], dstb[s],
                                  semidx[s]).wait()
            pltpu.async_copy(xl_hbm.at[srcb[s]], xlr[s], semrow[s])
            pltpu.async_copy(xr_hbm.at[dstb[s]], xrr[s], semrow[s])

        def rows_wait(s):
            pltpu.make_async_copy(xl_hbm.at[srcb[s]], xlr[s], semrow[s]).wait()
            pltpu.make_async_copy(xr_hbm.at[dstb[s]], xrr[s], semrow[s]).wait()

        def scat_wait(s):
            pltpu.make_async_copy(wbuf, acc.at[sidx[s]], semscat).wait()

        pltpu.sync_copy(src_hbm.at[pl.ds(wid * EW, K)], srcb[0])
        pltpu.sync_copy(dst_hbm.at[pl.ds(wid * EW, K)], dstb[0])
        pltpu.async_copy(xl_hbm.at[srcb[0]], xlr[0], semrow[0])
        pltpu.async_copy(xr_hbm.at[dstb[0]], xrr[0], semrow[0])
        idx_issue(1, 1)
        plsc.subcore_barrier()

        def compute(j, sA):
            xlrA, xrrA = xlr[sA], xrr[sA]

            def erow(e, _):
                xs = [xlrA[e, pl.ds(v * L, L)] for v in range(NV)]
                rs = [xrrA[e, pl.ds(v * L, L)] for v in range(NV)]
                lvec = zero16
                for h in range(H):
                    hs = zero16
                    for v in range(h * VPH, (h + 1) * VPH):
                        s = xs[v] + rs[v]
                        hs = hs + jnp.maximum(s, 0.2 * s) * att_vecs[v]
                    logit = jnp.sum(hs)
                    lvec = lvec + jnp.where(lane == h, logit, 0.0)
                pvec = jnp.where(lane < H, jnp.exp(lvec), 0.0)
                sp[e] = pvec
                for v in range(NV):
                    wbuf[e, pl.ds(v * L, L)] = xs[v] * pvec[v // VPH]
                return 0
            lax.fori_loop(0, K, erow, 0)

            def grp(g, _):
                eids = lane + g * L
                dvec = dstb[sA][pl.ds(g * L, L)]
                sidx[sA][pl.ds(g * L, L)] = dvec
                sidx[sA][pl.ds(K + g * L, L)] = (
                    NPAD + lax.shift_right_logical(dvec, 3))
                cols0 = jnp.bitwise_and(dvec, 7) * L
                czb[sA][pl.ds(g * L, L)] = cols0
                for h in range(H):
                    ph = plsc.load_gather(
                        sp, [eids, jnp.full((L,), h, jnp.int32)])
                    plsc.store_scatter(wbuf, [K + eids, cols0 + h], ph)
                return 0
            lax.fori_loop(0, K // L, grp, 0)

        def unpack_zero(sPrev):
            def gz(g, _):
                eids = lane + g * L
                cols0 = czb[sPrev][pl.ds(g * L, L)]
                for h in range(H):
                    plsc.store_scatter(wbuf, [K + eids, cols0 + h], zero16)
                return 0
            lax.fori_loop(0, K // L, gz, 0)

        def half(j, sA, sB, first, may_next, may_next2):
            if may_next:
                @pl.when(j + 1 < CH)
                def _():
                    rows_issue(sB)
            else:
                rows_issue(sB)
            rows_wait(sA)
            if first is not None:
                @pl.when(first)
                def _():
                    scat_wait(sB)
                    unpack_zero(sB)
            else:
                scat_wait(sB)
                unpack_zero(sB)
            compute(j, sA)
            pltpu.async_copy(wbuf, acc.at[sidx[sA]], semscat, add=True)
            if may_next2:
                @pl.when(j + 2 < CH)
                def _():
                    idx_issue(j + 2, sA)
            else:
                idx_issue(j + 2, sA)

        def step(jj, _):
            j = 2 * jj
            half(j, 0, 1, first=jj > 0, may_next=False, may_next2=True)
            half(j + 1, 1, 0, first=None, may_next=True, may_next2=True)
            return 0
        lax.fori_loop(0, CH // 2, step, 0)

        scat_wait(1)
        plsc.subcore_barrier()
        r0 = si * (NPAD // NS)
        pltpu.sync_copy(acc.at[pl.ds(r0, NPAD // NS)],
                        val_hbm.at[ci, pl.ds(r0, NPAD // NS)])
        d0 = si * DRPT
        pltpu.sync_copy(acc.at[pl.ds(NPAD + d0, DRPT)],
                        den_hbm.at[ci, pl.ds(d0, DRPT)])

    return pl.kernel(
        body,
        out_type=(jax.ShapeDtypeStruct((NC, NPAD, D), jnp.float32),
                  jax.ShapeDtypeStruct((NC, NPD8, D), jnp.float32)),
        mesh=mesh,
        compiler_params=pltpu.CompilerParams(needs_layout_passes=False),
        scratch_types=(
            pltpu.VMEM((K,), jnp.int32),
            pltpu.VMEM((K,), jnp.int32),
            pltpu.VMEM((K,), jnp.int32),
            pltpu.VMEM((K,), jnp.int32),
            pltpu.VMEM((K, D), jnp.float32),
            pltpu.VMEM((K, D), jnp.float32),
            pltpu.VMEM((K, D), jnp.float32),
            pltpu.VMEM((K, D), jnp.float32),
            pltpu.VMEM((2 * K,), jnp.int32),
            pltpu.VMEM((2 * K,), jnp.int32),
            pltpu.VMEM((K,), jnp.int32),
            pltpu.VMEM((K,), jnp.int32),
            pltpu.VMEM((2 * K, D), jnp.float32),
            pltpu.VMEM((K, L), jnp.float32),
            pltpu.VMEM((D,), jnp.float32),
            pltpu.VMEM_SHARED((NPAD + NPD8, D), jnp.float32),
            pltpu.SemaphoreType.DMA,
            pltpu.SemaphoreType.DMA,
            pltpu.SemaphoreType.DMA,
            pltpu.SemaphoreType.DMA,
            pltpu.SemaphoreType.DMA,
        ),
    )


@functools.lru_cache(maxsize=None)
def _sc_layer(H):
    return _make_sc_layer(H)


def kernel(x, edge_index, W_l1, W_r1, att1, b1, W_l2, W_r2, att2, b2):
    xpad = jnp.zeros((NPAD, D), jnp.float32).at[:N].set(x)
    loop = jnp.arange(N, dtype=jnp.int32)
    pad = jnp.full((ET_PAD - ET,), NPAD - 1, jnp.int32)
    src = jnp.concatenate([edge_index[0].astype(jnp.int32), loop, pad])
    dst = jnp.concatenate([edge_index[1].astype(jnp.int32), loop, pad])

    xl1, xr1 = _mm2(xpad, W_l1, W_r1)
    val1, den1 = _sc_layer(4)(xl1, xr1, src, dst, att1.reshape(D))
    xl2, xr2 = _combine1(val1, den1.reshape(NC, NPAD, L), b1, W_l2, W_r2)
    val2, den2 = _sc_layer(1)(xl2, xr2, src, dst, att2.reshape(D))
    out = _final(val2, den2.reshape(NC, NPAD, L), b2)
    return out[:N]


# speedup vs baseline: 58.0773x; 1.0282x over previous
import functools

import jax
import jax.numpy as jnp
from jax import lax
from jax.experimental import pallas as pl
from jax.experimental.pallas import tpu as pltpu
from jax.experimental.pallas import tpu_sc as plsc

N, E, D = 10000, 320000, 128
NPAD = 10240
ET = E + N
NC, NS, L = 2, 16, 16
NW = NC * NS
K = 32
EW = 10368
CH = EW // K
ET_PAD = NW * EW
ROWS_PER_TILE = NPAD // NS
NPD8 = NPAD // 8
DRPT = NPD8 // NS

BR = 1024


def _mm2_body(x_ref, wl_ref, wr_ref, out_ref):
    xb = x_ref[...]
    out_ref[0] = jnp.dot(xb, wl_ref[...], preferred_element_type=jnp.float32)
    out_ref[1] = jnp.dot(xb, wr_ref[...], preferred_element_type=jnp.float32)


def _mm2(xp, wl, wr):
    return pl.pallas_call(
        _mm2_body,
        grid=(NPAD // BR,),
        in_specs=[pl.BlockSpec((BR, D), lambda i: (i, 0)),
                  pl.BlockSpec((D, D), lambda i: (0, 0)),
                  pl.BlockSpec((D, D), lambda i: (0, 0))],
        out_specs=pl.BlockSpec((2, BR, D), lambda i: (0, i, 0)),
        out_shape=jax.ShapeDtypeStruct((2, NPAD, D), jnp.float32),
    )(xp, wl, wr)


def _combine1_body(val_ref, den_ref, b_ref, wl_ref, wr_ref, out_ref):
    v = val_ref[0] + val_ref[1]
    d = den_ref[0] + den_ref[1]
    cw = D // 4
    parts = [v[:, h * cw:(h + 1) * cw] / (d[:, h:h + 1] + 1e-16)
             for h in range(4)]
    h1 = jnp.concatenate(parts, axis=1) + b_ref[...]
    act = jnp.maximum(h1, 0.0) + jnp.log1p(jnp.exp(-jnp.abs(h1)))
    out_ref[0] = jnp.dot(act, wl_ref[...], preferred_element_type=jnp.float32)
    out_ref[1] = jnp.dot(act, wr_ref[...], preferred_element_type=jnp.float32)


def _combine1(val, den, b, wl, wr):
    return pl.pallas_call(
        _combine1_body,
        grid=(NPAD // BR,),
        in_specs=[pl.BlockSpec((2, BR, D), lambda i: (0, i, 0)),
                  pl.BlockSpec((2, BR, L), lambda i: (0, i, 0)),
                  pl.BlockSpec((1, D), lambda i: (0, 0)),
                  pl.BlockSpec((D, D), lambda i: (0, 0)),
                  pl.BlockSpec((D, D), lambda i: (0, 0))],
        out_specs=pl.BlockSpec((2, BR, D), lambda i: (0, i, 0)),
        out_shape=jax.ShapeDtypeStruct((2, NPAD, D), jnp.float32),
    )(val, den, b.reshape(1, D), wl, wr)


def _final_body(val_ref, den_ref, b_ref, out_ref):
    v = val_ref[0] + val_ref[1]
    d = den_ref[0] + den_ref[1]
    out_ref[...] = v / (d[:, 0:1] + 1e-16) + b_ref[...]


def _final(val, den, b):
    return pl.pallas_call(
        _final_body,
        grid=(NPAD // BR,),
        in_specs=[pl.BlockSpec((2, BR, D), lambda i: (0, i, 0)),
                  pl.BlockSpec((2, BR, L), lambda i: (0, i, 0)),
                  pl.BlockSpec((1, D), lambda i: (0, 0))],
        out_specs=pl.BlockSpec((BR, D), lambda i: (i, 0)),
        out_shape=jax.ShapeDtypeStruct((NPAD, D), jnp.float32),
    )(val, den, b.reshape(1, D))


def _make_sc_layer(H):
    C = D // H
    mesh = plsc.VectorSubcoreMesh(core_axis_name="c", subcore_axis_name="s",
                                  num_cores=NC, num_subcores=NS)

    def body(tbl_hbm, src_hbm, dsto_hbm, att_hbm, val_hbm, den_hbm,
             gidx0, gidx1, rows0, rows1, sidx0, sidx1, czb0, czb1,
             wbuf, sp, attv, acc,
             semidx0, semidx1, semrow0, semrow1, semscat):
        ci = lax.axis_index("c")
        si = lax.axis_index("s")
        wid = si * NC + ci

        zero16 = jnp.zeros((L,), jnp.float32)
        gidx = (gidx0, gidx1)
        rows = (rows0, rows1)
        sidx = (sidx0, sidx1)
        czb = (czb0, czb1)
        semidx = (semidx0, semidx1)
        semrow = (semrow0, semrow1)

        def zrow(r, _):
            for v in range(D // L):
                wbuf[r, pl.ds(v * L, L)] = zero16
            return 0
        lax.fori_loop(0, 2 * K, zrow, 0)
        def zsp(r, _):
            sp[r] = zero16
            return 0
        lax.fori_loop(0, K, zsp, 0)
        for i in range(12):
            r0 = si * 720 + i * 60
            pltpu.sync_copy(wbuf.at[pl.ds(0, 60)], acc.at[pl.ds(r0, 60)])
        pltpu.sync_copy(att_hbm, attv)

        att_vecs = [attv[pl.ds(k * L, L)] for k in range(D // L)]
        lane = lax.iota(jnp.int32, L)
        NV = D // L
        VPH = NV // H

        def idx_issue(jn, s):
            base = wid * EW + jn * K
            pltpu.async_copy(src_hbm.at[pl.ds(base, K)],
                             gidx[s].at[pl.ds(0, K)], semidx[s])
            pltpu.async_copy(dsto_hbm.at[pl.ds(base, K)],
                             gidx[s].at[pl.ds(K, K)], semidx[s])

        def rows_issue(s):
            pltpu.make_async_copy(src_hbm.at[pl.ds(0, K)],
                                  gidx[s].at[pl.ds(0, K)], semidx[s]).wait()
            pltpu.make_async_copy(dsto_hbm.at[pl.ds(0, K)],
                                  gidx[s].at[pl.ds(K, K)], semidx[s]).wait()
            pltpu.async_copy(tbl_hbm.at[gidx[s]], rows[s], semrow[s])

        def rows_wait(s):
            pltpu.make_async_copy(tbl_hbm.at[gidx[s]], rows[s],
                                  semrow[s]).wait()

        def scat_wait(s):
            pltpu.make_async_copy(wbuf, acc.at[sidx[s]], semscat).wait()

        pltpu.sync_copy(src_hbm.at[pl.ds(wid * EW, K)], gidx[0].at[pl.ds(0, K)])
        pltpu.sync_copy(dsto_hbm.at[pl.ds(wid * EW, K)],
                        gidx[0].at[pl.ds(K, K)])
        pltpu.async_copy(tbl_hbm.at[gidx[0]], rows[0], semrow[0])
        idx_issue(1, 1)
        plsc.subcore_barrier()

        def compute(j, sA):
            rowsA = rows[sA]

            @plsc.parallel_loop(0, K, unroll=2)
            def erow(e):
                xs = [rowsA[e, pl.ds(v * L, L)] for v in range(NV)]
                rs = [rowsA[K + e, pl.ds(v * L, L)] for v in range(NV)]
                lvec = zero16
                for h in range(H):
                    hs = zero16
                    for v in range(h * VPH, (h + 1) * VPH):
                        s = xs[v] + rs[v]
                        hs = hs + jnp.maximum(s, 0.2 * s) * att_vecs[v]
                    logit = jnp.sum(hs)
                    lvec = lvec + jnp.where(lane == h, logit, 0.0)
                pvec = jnp.exp(lvec)
                sp[e] = pvec
                for v in range(NV):
                    wbuf[e, pl.ds(v * L, L)] = xs[v] * pvec[v // VPH]

            @plsc.parallel_loop(0, K // L, unroll=1)
            def grp(g):
                eids = lane + g * L
                dvec = gidx[sA][pl.ds(K + g * L, L)] - NPAD
                sidx[sA][pl.ds(g * L, L)] = dvec
                sidx[sA][pl.ds(K + g * L, L)] = (
                    NPAD + lax.shift_right_logical(dvec, 3))
                cols0 = jnp.bitwise_and(dvec, 7) * L
                czb[sA][pl.ds(g * L, L)] = cols0
                for h in range(H):
                    ph = plsc.load_gather(
                        sp, [eids, jnp.full((L,), h, jnp.int32)])
                    plsc.store_scatter(wbuf, [K + eids, cols0 + h], ph)

        def unpack_zero(sPrev):
            @plsc.parallel_loop(0, K // L, unroll=1)
            def gz(g):
                eids = lane + g * L
                cols0 = czb[sPrev][pl.ds(g * L, L)]
                for h in range(H):
                    plsc.store_scatter(wbuf, [K + eids, cols0 + h], zero16)

        def half(j, sA, sB, first, may_next, may_next2):
            if may_next:
                @pl.when(j + 1 < CH)
                def _():
                    rows_issue(sB)
            else:
                rows_issue(sB)
            rows_wait(sA)
            if first is not None:
                @pl.when(first)
                def _():
                    scat_wait(sB)
                    unpack_zero(sB)
            else:
                scat_wait(sB)
                unpack_zero(sB)
            compute(j, sA)
            pltpu.async_copy(wbuf, acc.at[sidx[sA]], semscat, add=True)
            if may_next2:
                @pl.when(j + 2 < CH)
                def _():
                    idx_issue(j + 2, sA)
            else:
                idx_issue(j + 2, sA)

        def step(jj, _):
            j = 2 * jj
            half(j, 0, 1, first=jj > 0, may_next=False, may_next2=True)
            half(j + 1, 1, 0, first=None, may_next=True, may_next2=True)
            return 0
        lax.fori_loop(0, CH // 2, step, 0)

        scat_wait(1)
        plsc.subcore_barrier()
        r0 = si * (NPAD // NS)
        pltpu.sync_copy(acc.at[pl.ds(r0, NPAD // NS)],
                        val_hbm.at[ci, pl.ds(r0, NPAD // NS)])
        d0 = si * DRPT
        pltpu.sync_copy(acc.at[pl.ds(NPAD + d0, DRPT)],
                        den_hbm.at[ci, pl.ds(d0, DRPT)])

    return pl.kernel(
        body,
        out_type=(jax.ShapeDtypeStruct((NC, NPAD, D), jnp.float32),
                  jax.ShapeDtypeStruct((NC, NPD8, D), jnp.float32)),
        mesh=mesh,
        compiler_params=pltpu.CompilerParams(needs_layout_passes=False),
        scratch_types=(
            pltpu.VMEM((2 * K,), jnp.int32),
            pltpu.VMEM((2 * K,), jnp.int32),
            pltpu.VMEM((2 * K, D), jnp.float32),
            pltpu.VMEM((2 * K, D), jnp.float32),
            pltpu.VMEM((2 * K,), jnp.int32),
            pltpu.VMEM((2 * K,), jnp.int32),
            pltpu.VMEM((K,), jnp.int32),
            pltpu.VMEM((K,), jnp.int32),
            pltpu.VMEM((2 * K, D), jnp.float32),
            pltpu.VMEM((K, L), jnp.float32),
            pltpu.VMEM((D,), jnp.float32),
            pltpu.VMEM_SHARED((NPAD + NPD8, D), jnp.float32),
            pltpu.SemaphoreType.DMA,
            pltpu.SemaphoreType.DMA,
            pltpu.SemaphoreType.DMA,
            pltpu.SemaphoreType.DMA,
            pltpu.SemaphoreType.DMA,
        ),
    )


@functools.lru_cache(maxsize=None)
def _sc_layer(H):
    return _make_sc_layer(H)


def kernel(x, edge_index, W_l1, W_r1, att1, b1, W_l2, W_r2, att2, b2):
    xpad = jnp.zeros((NPAD, D), jnp.float32).at[:N].set(x)
    loop = jnp.arange(N, dtype=jnp.int32)
    pad = jnp.full((ET_PAD - ET,), NPAD - 1, jnp.int32)
    src = jnp.concatenate([edge_index[0].astype(jnp.int32), loop, pad])
    dst = jnp.concatenate([edge_index[1].astype(jnp.int32), loop, pad])

    dsto = dst + NPAD
    tbl1 = _mm2(xpad, W_l1, W_r1).reshape(2 * NPAD, D)
    val1, den1 = _sc_layer(4)(tbl1, src, dsto, att1.reshape(D))
    tbl2 = _combine1(val1, den1.reshape(NC, NPAD, L), b1,
                     W_l2, W_r2).reshape(2 * NPAD, D)
    val2, den2 = _sc_layer(1)(tbl2, src, dsto, att2.reshape(D))
    out = _final(val2, den2.reshape(NC, NPAD, L), b2)
    return out[:N]


# speedup vs baseline: 59.1193x; 1.0179x over previous
import functools

import jax
import jax.numpy as jnp
from jax import lax
from jax.experimental import pallas as pl
from jax.experimental.pallas import tpu as pltpu
from jax.experimental.pallas import tpu_sc as plsc

N, E, D = 10000, 320000, 128
NPAD = 10240
ET = E + N
NC, NS, L = 2, 16, 16
NW = NC * NS
K = 32
EW = 10368
CH = EW // K
ET_PAD = NW * EW
ROWS_PER_TILE = NPAD // NS
NPD8 = NPAD // 8
DRPT = NPD8 // NS

BR = 1024


def _mm2_body(x_ref, wl_ref, wr_ref, out_ref):
    xb = x_ref[...]
    out_ref[0] = jnp.dot(xb, wl_ref[...], preferred_element_type=jnp.float32)
    out_ref[1] = jnp.dot(xb, wr_ref[...], preferred_element_type=jnp.float32)


def _mm2(xp, wl, wr):
    return pl.pallas_call(
        _mm2_body,
        grid=(NPAD // BR,),
        in_specs=[pl.BlockSpec((BR, D), lambda i: (i, 0)),
                  pl.BlockSpec((D, D), lambda i: (0, 0)),
                  pl.BlockSpec((D, D), lambda i: (0, 0))],
        out_specs=pl.BlockSpec((2, BR, D), lambda i: (0, i, 0)),
        out_shape=jax.ShapeDtypeStruct((2, NPAD, D), jnp.float32),
    )(xp, wl, wr)


def _combine1_body(val_ref, den_ref, b_ref, wl_ref, wr_ref, out_ref):
    v = val_ref[0] + val_ref[1]
    d = den_ref[0] + den_ref[1]
    cw = D // 4
    parts = [v[:, h * cw:(h + 1) * cw] / (d[:, h:h + 1] + 1e-16)
             for h in range(4)]
    h1 = jnp.concatenate(parts, axis=1) + b_ref[...]
    act = jnp.maximum(h1, 0.0) + jnp.log1p(jnp.exp(-jnp.abs(h1)))
    out_ref[0] = jnp.dot(act, wl_ref[...], preferred_element_type=jnp.float32)
    out_ref[1] = jnp.dot(act, wr_ref[...], preferred_element_type=jnp.float32)


def _combine1(val, den, b, wl, wr):
    return pl.pallas_call(
        _combine1_body,
        grid=(NPAD // BR,),
        in_specs=[pl.BlockSpec((2, BR, D), lambda i: (0, i, 0)),
                  pl.BlockSpec((2, BR, L), lambda i: (0, i, 0)),
                  pl.BlockSpec((1, D), lambda i: (0, 0)),
                  pl.BlockSpec((D, D), lambda i: (0, 0)),
                  pl.BlockSpec((D, D), lambda i: (0, 0))],
        out_specs=pl.BlockSpec((2, BR, D), lambda i: (0, i, 0)),
        out_shape=jax.ShapeDtypeStruct((2, NPAD, D), jnp.float32),
    )(val, den, b.reshape(1, D), wl, wr)


def _final_body(val_ref, den_ref, b_ref, out_ref):
    v = val_ref[0] + val_ref[1]
    d = den_ref[0] + den_ref[1]
    out_ref[...] = v / (d[:, 0:1] + 1e-16) + b_ref[...]


def _final(val, den, b):
    return pl.pallas_call(
        _final_body,
        grid=(NPAD // BR,),
        in_specs=[pl.BlockSpec((2, BR, D), lambda i: (0, i, 0)),
                  pl.BlockSpec((2, BR, L), lambda i: (0, i, 0)),
                  pl.BlockSpec((1, D), lambda i: (0, 0))],
        out_specs=pl.BlockSpec((BR, D), lambda i: (i, 0)),
        out_shape=jax.ShapeDtypeStruct((NPAD, D), jnp.float32),
    )(val, den, b.reshape(1, D))


def _make_sc_layer(H):
    C = D // H
    mesh = plsc.VectorSubcoreMesh(core_axis_name="c", subcore_axis_name="s",
                                  num_cores=NC, num_subcores=NS)

    def body(tbl_hbm, src_hbm, dsto_hbm, att_hbm, val_hbm, den_hbm,
             gidx0, gidx1, gidx2, rows0, rows1, rows2, sidx0, sidx1, sidx2,
             czb0, czb1, czb2, wbuf, sp, attv, acc,
             semidx0, semidx1, semidx2, semrow0, semrow1, semrow2, semscat):
        ci = lax.axis_index("c")
        si = lax.axis_index("s")
        wid = si * NC + ci

        zero16 = jnp.zeros((L,), jnp.float32)
        gidx = (gidx0, gidx1, gidx2)
        rows = (rows0, rows1, rows2)
        sidx = (sidx0, sidx1, sidx2)
        czb = (czb0, czb1, czb2)
        semidx = (semidx0, semidx1, semidx2)
        semrow = (semrow0, semrow1, semrow2)

        def zrow(r, _):
            for v in range(D // L):
                wbuf[r, pl.ds(v * L, L)] = zero16
            return 0
        lax.fori_loop(0, 2 * K, zrow, 0)
        def zsp(r, _):
            sp[r] = zero16
            return 0
        lax.fori_loop(0, K, zsp, 0)
        for i in range(12):
            r0 = si * 720 + i * 60
            pltpu.sync_copy(wbuf.at[pl.ds(0, 60)], acc.at[pl.ds(r0, 60)])
        pltpu.sync_copy(att_hbm, attv)

        att_vecs = [attv[pl.ds(k * L, L)] for k in range(D // L)]
        lane = lax.iota(jnp.int32, L)
        NV = D // L
        VPH = NV // H

        def idx_issue(jn, s):
            base = wid * EW + jn * K
            pltpu.async_copy(src_hbm.at[pl.ds(base, K)],
                             gidx[s].at[pl.ds(0, K)], semidx[s])
            pltpu.async_copy(dsto_hbm.at[pl.ds(base, K)],
                             gidx[s].at[pl.ds(K, K)], semidx[s])

        def rows_issue(s):
            pltpu.make_async_copy(src_hbm.at[pl.ds(0, K)],
                                  gidx[s].at[pl.ds(0, K)], semidx[s]).wait()
            pltpu.make_async_copy(dsto_hbm.at[pl.ds(0, K)],
                                  gidx[s].at[pl.ds(K, K)], semidx[s]).wait()
            pltpu.async_copy(tbl_hbm.at[gidx[s]], rows[s], semrow[s])

        def rows_wait(s):
            pltpu.make_async_copy(tbl_hbm.at[gidx[s]], rows[s],
                                  semrow[s]).wait()

        def scat_wait(s):
            pltpu.make_async_copy(wbuf, acc.at[sidx[s]], semscat).wait()

        pltpu.sync_copy(src_hbm.at[pl.ds(wid * EW, K)], gidx[0].at[pl.ds(0, K)])
        pltpu.sync_copy(dsto_hbm.at[pl.ds(wid * EW, K)],
                        gidx[0].at[pl.ds(K, K)])
        pltpu.async_copy(tbl_hbm.at[gidx[0]], rows[0], semrow[0])
        idx_issue(1, 1)
        rows_issue(1)
        idx_issue(2, 2)
        plsc.subcore_barrier()

        def compute(j, sA):
            rowsA = rows[sA]

            @plsc.parallel_loop(0, K, unroll=2)
            def erow(e):
                xs = [rowsA[e, pl.ds(v * L, L)] for v in range(NV)]
                rs = [rowsA[K + e, pl.ds(v * L, L)] for v in range(NV)]
                lvec = zero16
                for h in range(H):
                    hs = zero16
                    for v in range(h * VPH, (h + 1) * VPH):
                        s = xs[v] + rs[v]
                        hs = hs + jnp.maximum(s, 0.2 * s) * att_vecs[v]
                    logit = jnp.sum(hs)
                    lvec = lvec + jnp.where(lane == h, logit, 0.0)
                pvec = jnp.exp(lvec)
                sp[e] = pvec
                for v in range(NV):
                    wbuf[e, pl.ds(v * L, L)] = xs[v] * pvec[v // VPH]

            @plsc.parallel_loop(0, K // L, unroll=1)
            def grp(g):
                eids = lane + g * L
                dvec = gidx[sA][pl.ds(K + g * L, L)] - NPAD
                sidx[sA][pl.ds(g * L, L)] = dvec
                sidx[sA][pl.ds(K + g * L, L)] = (
                    NPAD + lax.shift_right_logical(dvec, 3))
                cols0 = jnp.bitwise_and(dvec, 7) * L
                czb[sA][pl.ds(g * L, L)] = cols0
                for h in range(H):
                    ph = plsc.load_gather(
                        sp, [eids, jnp.full((L,), h, jnp.int32)])
                    plsc.store_scatter(wbuf, [K + eids, cols0 + h], ph)

        def unpack_zero(sPrev):
            @plsc.parallel_loop(0, K // L, unroll=1)
            def gz(g):
                eids = lane + g * L
                cols0 = czb[sPrev][pl.ds(g * L, L)]
                for h in range(H):
                    plsc.store_scatter(wbuf, [K + eids, cols0 + h], zero16)

        def half(j, r, first):
            sPrev = (r + 2) % 3
            @pl.when(j + 2 < CH)
            def _():
                rows_issue((r + 2) % 3)
            rows_wait(r)
            if first is not None:
                @pl.when(first)
                def _():
                    scat_wait(sPrev)
                    unpack_zero(sPrev)
            else:
                scat_wait(sPrev)
                unpack_zero(sPrev)
            compute(j, r)
            pltpu.async_copy(wbuf, acc.at[sidx[r]], semscat, add=True)
            @pl.when(j + 3 < CH)
            def _():
                idx_issue(j + 3, r)

        def step(jj, _):
            j = 3 * jj
            half(j, 0, first=jj > 0)
            half(j + 1, 1, first=None)
            half(j + 2, 2, first=None)
            return 0
        lax.fori_loop(0, CH // 3, step, 0)

        scat_wait((CH - 1) % 3)
        plsc.subcore_barrier()
        r0 = si * (NPAD // NS)
        pltpu.sync_copy(acc.at[pl.ds(r0, NPAD // NS)],
                        val_hbm.at[ci, pl.ds(r0, NPAD // NS)])
        d0 = si * DRPT
        pltpu.sync_copy(acc.at[pl.ds(NPAD + d0, DRPT)],
                        den_hbm.at[ci, pl.ds(d0, DRPT)])

    return pl.kernel(
        body,
        out_type=(jax.ShapeDtypeStruct((NC, NPAD, D), jnp.float32),
                  jax.ShapeDtypeStruct((NC, NPD8, D), jnp.float32)),
        mesh=mesh,
        compiler_params=pltpu.CompilerParams(needs_layout_passes=False),
        scratch_types=(
            pltpu.VMEM((2 * K,), jnp.int32),
            pltpu.VMEM((2 * K,), jnp.int32),
            pltpu.VMEM((2 * K,), jnp.int32),
            pltpu.VMEM((2 * K, D), jnp.float32),
            pltpu.VMEM((2 * K, D), jnp.float32),
            pltpu.VMEM((2 * K, D), jnp.float32),
            pltpu.VMEM((2 * K,), jnp.int32),
            pltpu.VMEM((2 * K,), jnp.int32),
            pltpu.VMEM((2 * K,), jnp.int32),
            pltpu.VMEM((K,), jnp.int32),
            pltpu.VMEM((K,), jnp.int32),
            pltpu.VMEM((K,), jnp.int32),
            pltpu.VMEM((2 * K, D), jnp.float32),
            pltpu.VMEM((K, L), jnp.float32),
            pltpu.VMEM((D,), jnp.float32),
            pltpu.VMEM_SHARED((NPAD + NPD8, D), jnp.float32),
            pltpu.SemaphoreType.DMA,
            pltpu.SemaphoreType.DMA,
            pltpu.SemaphoreType.DMA,
            pltpu.SemaphoreType.DMA,
            pltpu.SemaphoreType.DMA,
            pltpu.SemaphoreType.DMA,
            pltpu.SemaphoreType.DMA,
        ),
    )


@functools.lru_cache(maxsize=None)
def _sc_layer(H):
    return _make_sc_layer(H)


def kernel(x, edge_index, W_l1, W_r1, att1, b1, W_l2, W_r2, att2, b2):
    xpad = jnp.zeros((NPAD, D), jnp.float32).at[:N].set(x)
    loop = jnp.arange(N, dtype=jnp.int32)
    pad = jnp.full((ET_PAD - ET,), NPAD - 1, jnp.int32)
    src = jnp.concatenate([edge_index[0].astype(jnp.int32), loop, pad])
    dst = jnp.concatenate([edge_index[1].astype(jnp.int32), loop, pad])

    dsto = dst + NPAD
    tbl1 = _mm2(xpad, W_l1, W_r1).reshape(2 * NPAD, D)
    val1, den1 = _sc_layer(4)(tbl1, src, dsto, att1.reshape(D))
    tbl2 = _combine1(val1, den1.reshape(NC, NPAD, L), b1,
                     W_l2, W_r2).reshape(2 * NPAD, D)
    val2, den2 = _sc_layer(1)(tbl2, src, dsto, att2.reshape(D))
    out = _final(val2, den2.reshape(NC, NPAD, L), b2)
    return out[:N]
